# baseline (speedup 1.0000x reference)
import functools
import math

import jax
import jax.numpy as jnp
from jax import lax
from jax.experimental import pallas as pl
from jax.experimental.pallas import tpu as pltpu
from jax.experimental.pallas import tpu_sc as plsc

_L = 16


def _sc_num_workers():
    info = plsc.get_sparse_core_info()
    return info.num_cores, info.num_subcores


def kernel(input_ids, table, offsets):
    B, F = input_ids.shape
    V, D = table.shape
    NC, NS = _sc_num_workers()
    NW = NC * NS
    assert B % NW == 0
    RPW = B // NW
    IPW = RPW * F
    assert IPW % _L == 0

    GCH = 128
    assert IPW % GCH == 0
    NCHUNK = IPW // GCH
    SPG = 13
    assert NCHUNK % SPG == 0
    NGROUP = NCHUNK // SPG
    GROUP_ROWS = SPG * GCH

    ids_flat = input_ids.astype(jnp.int32).reshape(B * F)
    assert V % 4 == 0 and 4 * D == 128 and D == 32
    W = 128
    n_chunks = (V + W - 1) // W
    n_steps = (n_chunks + 3) // 4
    QP = n_steps * W
    VP = 4 * QP

    def _relayout(i0, i1, i2, i3, out_ref):
        out_ref[...] = jnp.concatenate(
            [i0[...].T, i1[...].T, i2[...].T, i3[...].T], axis=1)

    uspec = lambda u: pl.BlockSpec(
        (D, W), lambda j, _u=u: (0, jnp.minimum(4 * j + _u, n_chunks - 1)))
    tableT = table.T
    table_g = pl.pallas_call(
        _relayout,
        grid=(n_steps,),
        in_specs=[uspec(0), uspec(1), uspec(2), uspec(3)],
        out_specs=pl.BlockSpec((W, 4 * D), lambda j: (j, 0)),
        out_shape=jax.ShapeDtypeStruct((QP, 4 * D), jnp.float32),
    )(tableT, tableT, tableT, tableT)
    table_lin = table_g.reshape(VP, D)
    PERIOD = F * _L // math.gcd(F, _L)
    REP = PERIOD // F
    NPAT = PERIOD // _L
    assert IPW % PERIOD == 0
    off_pat = jnp.tile(offsets.astype(jnp.int32), REP)

    mesh = plsc.VectorSubcoreMesh(core_axis_name="c", subcore_axis_name="s")

    @functools.partial(
        pl.kernel,
        mesh=mesh,
        compiler_params=pltpu.CompilerParams(use_tc_tiling_on_sc=False),
        out_type=jax.ShapeDtypeStruct((B * F, D), jnp.float32),
        scratch_types=[
            pltpu.VMEM((IPW,), jnp.int32),
            pltpu.VMEM((PERIOD,), jnp.int32),
            pltpu.VMEM((GROUP_ROWS, D), jnp.float32),
            pltpu.SemaphoreType.DMA,
        ],
    )
    def _k(ids_hbm, table_hbm, off_hbm, out_hbm, idx_v, off_v, buf_v, sem):
        wid = lax.axis_index("s") * NC + lax.axis_index("c")
        base = wid * IPW

        pltpu.sync_copy(off_hbm, off_v)
        pltpu.sync_copy(ids_hbm.at[pl.ds(base, IPW)], idx_v)

        def add_off(j, _):
            off = off_v[pl.ds(lax.rem(j, NPAT) * _L, _L)]
            sl = pl.ds(j * _L, _L)
            i = idx_v[sl] + off
            idx_v[sl] = (((i >> 9) << 9)
                         | ((i & (W - 1)) << 2)
                         | ((i >> 7) & 3))
            return _

        lax.fori_loop(0, IPW // _L, add_off, None)

        def group(g, _):
            gbase = g * GROUP_ROWS
            copies = []
            for i in range(SPG):
                cbase = gbase + i * GCH
                copies.append(pltpu.async_copy(
                    table_hbm.at[idx_v.at[pl.ds(cbase, GCH)]],
                    buf_v.at[pl.ds(i * GCH, GCH)],
                    sem,
                ))
            for c in copies:
                c.wait()
            pltpu.sync_copy(buf_v, out_hbm.at[pl.ds(base + gbase, GROUP_ROWS)])
            return _

        lax.fori_loop(0, NGROUP, group, None)

    out = _k(ids_flat, table_lin, off_pat)
    return out.reshape(B, F, D)


# speedup vs baseline: 3.4958x; 3.4958x over previous
import functools
import math

import jax
import jax.numpy as jnp
from jax import lax
from jax.experimental import pallas as pl
from jax.experimental.pallas import tpu as pltpu
from jax.experimental.pallas import tpu_sc as plsc

_L = 16


def _sc_num_workers():
    info = plsc.get_sparse_core_info()
    return info.num_cores, info.num_subcores


def kernel(input_ids, table, offsets):
    B, F = input_ids.shape
    V, D = table.shape
    NC, NS = _sc_num_workers()
    NW = NC * NS
    assert B % NW == 0
    RPW = B // NW
    IPW = RPW * F
    assert IPW % _L == 0

    GCH = 128
    assert IPW % GCH == 0
    NCHUNK = IPW // GCH
    SPG = 13
    assert NCHUNK % SPG == 0
    NGROUP = NCHUNK // SPG
    GROUP_ROWS = SPG * GCH

    ids_flat = input_ids.astype(jnp.int32).reshape(B * F)
    assert V % 4 == 0 and 4 * D == 128 and D == 32
    W = 37 * 128
    lanes_pad = 128 * ((V + 127) // 128)
    n_chunks = (V + W - 1) // W
    assert n_chunks * W == lanes_pad
    n_steps = (n_chunks + 3) // 4
    QP = n_steps * W
    VP = 4 * QP

    def _relayout(i0, i1, i2, i3, out_ref):
        out_ref[...] = jnp.concatenate(
            [i0[...].T, i1[...].T, i2[...].T, i3[...].T], axis=1)

    uspec = lambda u: pl.BlockSpec(
        (D, W), lambda j, _u=u: (0, jnp.minimum(4 * j + _u, n_chunks - 1)))
    tableT = table.T
    table_g = pl.pallas_call(
        _relayout,
        grid=(n_steps,),
        in_specs=[uspec(0), uspec(1), uspec(2), uspec(3)],
        out_specs=pl.BlockSpec((W, 4 * D), lambda j: (j, 0)),
        out_shape=jax.ShapeDtypeStruct((QP, 4 * D), jnp.float32),
    )(tableT, tableT, tableT, tableT)
    table_lin = table_g.reshape(VP, D)
    PERIOD = F * _L // math.gcd(F, _L)
    REP = PERIOD // F
    NPAT = PERIOD // _L
    assert IPW % PERIOD == 0
    off_pat = jnp.tile(offsets.astype(jnp.int32), REP)

    mesh = plsc.VectorSubcoreMesh(core_axis_name="c", subcore_axis_name="s")

    @functools.partial(
        pl.kernel,
        mesh=mesh,
        compiler_params=pltpu.CompilerParams(use_tc_tiling_on_sc=False),
        out_type=jax.ShapeDtypeStruct((B * F, D), jnp.float32),
        scratch_types=[
            pltpu.VMEM((IPW,), jnp.int32),
            pltpu.VMEM((PERIOD,), jnp.int32),
            pltpu.VMEM((GROUP_ROWS, D), jnp.float32),
            pltpu.SemaphoreType.DMA,
        ],
    )
    def _k(ids_hbm, table_hbm, off_hbm, out_hbm, idx_v, off_v, buf_v, sem):
        wid = lax.axis_index("s") * NC + lax.axis_index("c")
        base = wid * IPW

        pltpu.sync_copy(off_hbm, off_v)
        pltpu.sync_copy(ids_hbm.at[pl.ds(base, IPW)], idx_v)

        def add_off(j, _):
            off = off_v[pl.ds(lax.rem(j, NPAT) * _L, _L)]
            sl = pl.ds(j * _L, _L)
            i = idx_v[sl] + off
            c = ((i.astype(jnp.float32) + 0.5)
                 * jnp.float32(1.0 / W)).astype(jnp.int32)
            w = i - c * W
            idx_v[sl] = ((((c >> 2) * W + w) << 2) | (c & 3))
            return _

        lax.fori_loop(0, IPW // _L, add_off, None)

        def group(g, _):
            gbase = g * GROUP_ROWS
            copies = []
            for i in range(SPG):
                cbase = gbase + i * GCH
                copies.append(pltpu.async_copy(
                    table_hbm.at[idx_v.at[pl.ds(cbase, GCH)]],
                    buf_v.at[pl.ds(i * GCH, GCH)],
                    sem,
                ))
            for c in copies:
                c.wait()
            pltpu.sync_copy(buf_v, out_hbm.at[pl.ds(base + gbase, GROUP_ROWS)])
            return _

        lax.fori_loop(0, NGROUP, group, None)

    out = _k(ids_flat, table_lin, off_pat)
    return out.reshape(B, F, D)


# speedup vs baseline: 4.6718x; 1.3364x over previous
import functools
import math

import jax
import jax.numpy as jnp
from jax import lax
from jax.experimental import pallas as pl
from jax.experimental.pallas import tpu as pltpu
from jax.experimental.pallas import tpu_sc as plsc

_L = 16


def _sc_num_workers():
    info = plsc.get_sparse_core_info()
    return info.num_cores, info.num_subcores


def kernel(input_ids, table, offsets):
    B, F = input_ids.shape
    V, D = table.shape
    NC, NS = _sc_num_workers()
    NW = NC * NS
    assert B % NW == 0
    RPW = B // NW
    IPW = RPW * F
    assert IPW % _L == 0

    GCH = 128
    assert IPW % GCH == 0
    NCHUNK = IPW // GCH
    SPG = 13
    assert NCHUNK % SPG == 0
    NGROUP = NCHUNK // SPG
    GROUP_ROWS = SPG * GCH

    ids_flat = input_ids.astype(jnp.int32).reshape(B * F)
    assert V % 4 == 0 and 4 * D == 128 and D == 32
    W = 37 * 128
    lanes_pad = 128 * ((V + 127) // 128)
    n_chunks = (V + W - 1) // W
    assert n_chunks * W == lanes_pad
    n_steps = (n_chunks + 3) // 4
    QP = n_steps * W
    VP = 4 * QP

    def _relayout(i0, i1, i2, i3, out_ref):
        col = lax.broadcasted_iota(jnp.int32, (D, 4 * D), 1)
        row = lax.broadcasted_iota(jnp.int32, (D, 4 * D), 0)
        acc = None
        for a, ref in enumerate((i0, i1, i2, i3)):
            p = (col == row + D * a).astype(jnp.float32)
            y = lax.dot_general(ref[...], p, (((0,), (0,)), ((), ())),
                                preferred_element_type=jnp.float32)
            acc = y if acc is None else acc + y
        out_ref[...] = acc

    uspec = lambda u: pl.BlockSpec(
        (D, W), lambda j, _u=u: (0, jnp.minimum(4 * j + _u, n_chunks - 1)))
    tableT = table.T
    table_g = pl.pallas_call(
        _relayout,
        grid=(n_steps,),
        in_specs=[uspec(0), uspec(1), uspec(2), uspec(3)],
        out_specs=pl.BlockSpec((W, 4 * D), lambda j: (j, 0)),
        out_shape=jax.ShapeDtypeStruct((QP, 4 * D), jnp.float32),
    )(tableT, tableT, tableT, tableT)
    table_lin = table_g.reshape(VP, D)
    PERIOD = F * _L // math.gcd(F, _L)
    REP = PERIOD // F
    NPAT = PERIOD // _L
    assert IPW % PERIOD == 0
    off_pat = jnp.tile(offsets.astype(jnp.int32), REP)

    mesh = plsc.VectorSubcoreMesh(core_axis_name="c", subcore_axis_name="s")

    @functools.partial(
        pl.kernel,
        mesh=mesh,
        compiler_params=pltpu.CompilerParams(use_tc_tiling_on_sc=False),
        out_type=jax.ShapeDtypeStruct((B * F, D), jnp.float32),
        scratch_types=[
            pltpu.VMEM((IPW,), jnp.int32),
            pltpu.VMEM((PERIOD,), jnp.int32),
            pltpu.VMEM((GROUP_ROWS, D), jnp.float32),
            pltpu.SemaphoreType.DMA,
        ],
    )
    def _k(ids_hbm, table_hbm, off_hbm, out_hbm, idx_v, off_v, buf_v, sem):
        wid = lax.axis_index("s") * NC + lax.axis_index("c")
        base = wid * IPW

        pltpu.sync_copy(off_hbm, off_v)
        pltpu.sync_copy(ids_hbm.at[pl.ds(base, IPW)], idx_v)

        def add_off(j, _):
            off = off_v[pl.ds(lax.rem(j, NPAT) * _L, _L)]
            sl = pl.ds(j * _L, _L)
            i = idx_v[sl] + off
            c = ((i.astype(jnp.float32) + 0.5)
                 * jnp.float32(1.0 / W)).astype(jnp.int32)
            w = i - c * W
            idx_v[sl] = ((((c >> 2) * W + w) << 2) | (c & 3))
            return _

        lax.fori_loop(0, IPW // _L, add_off, None)

        def group(g, _):
            gbase = g * GROUP_ROWS
            copies = []
            for i in range(SPG):
                cbase = gbase + i * GCH
                copies.append(pltpu.async_copy(
                    table_hbm.at[idx_v.at[pl.ds(cbase, GCH)]],
                    buf_v.at[pl.ds(i * GCH, GCH)],
                    sem,
                ))
            for c in copies:
                c.wait()
            pltpu.sync_copy(buf_v, out_hbm.at[pl.ds(base + gbase, GROUP_ROWS)])
            return _

        lax.fori_loop(0, NGROUP, group, None)

    out = _k(ids_flat, table_lin, off_pat)
    return out.reshape(B, F, D)


# speedup vs baseline: 5.5209x; 1.1817x over previous
import functools
import math

import jax
import jax.numpy as jnp
from jax import lax
from jax.experimental import pallas as pl
from jax.experimental.pallas import tpu as pltpu
from jax.experimental.pallas import tpu_sc as plsc

_L = 16


def _sc_num_workers():
    info = plsc.get_sparse_core_info()
    return info.num_cores, info.num_subcores


def kernel(input_ids, table, offsets):
    B, F = input_ids.shape
    V, D = table.shape
    NC, NS = _sc_num_workers()
    NW = NC * NS
    assert B % NW == 0
    RPW = B // NW
    IPW = RPW * F
    assert IPW % _L == 0

    GCH = 128
    assert IPW % GCH == 0
    NCHUNK = IPW // GCH
    SPG = 13
    assert NCHUNK % SPG == 0
    NGROUP = NCHUNK // SPG
    GROUP_ROWS = SPG * GCH

    ids_flat = input_ids.astype(jnp.int32).reshape(B * F)
    assert V % 4 == 0 and 4 * D == 128 and D == 32
    W = 37 * 128
    lanes_pad = 128 * ((V + 127) // 128)
    n_chunks = (V + W - 1) // W
    assert n_chunks * W == lanes_pad
    n_steps = (n_chunks + 3) // 4
    QP = n_steps * W
    VP = 4 * QP

    def _relayout(i0, i1, i2, i3, out_ref):
        x = jnp.concatenate([i0[...], i1[...], i2[...], i3[...]], axis=0)
        col = lax.broadcasted_iota(jnp.int32, (4 * D, 4 * D), 1)
        row = lax.broadcasted_iota(jnp.int32, (4 * D, 4 * D), 0)
        ident = (col == row).astype(jnp.bfloat16)
        x0 = x.astype(jnp.bfloat16)
        r1 = x - x0.astype(jnp.float32)
        x1 = r1.astype(jnp.bfloat16)
        x2 = (r1 - x1.astype(jnp.float32)).astype(jnp.bfloat16)
        acc = None
        for xk in (x0, x1, x2):
            y = lax.dot_general(xk, ident, (((0,), (0,)), ((), ())),
                                preferred_element_type=jnp.float32)
            acc = y if acc is None else acc + y
        out_ref[...] = acc

    uspec = lambda u: pl.BlockSpec(
        (D, W), lambda j, _u=u: (0, jnp.minimum(4 * j + _u, n_chunks - 1)))
    tableT = table.T
    table_g = pl.pallas_call(
        _relayout,
        grid=(n_steps,),
        in_specs=[uspec(0), uspec(1), uspec(2), uspec(3)],
        out_specs=pl.BlockSpec((W, 4 * D), lambda j: (j, 0)),
        out_shape=jax.ShapeDtypeStruct((QP, 4 * D), jnp.float32),
    )(tableT, tableT, tableT, tableT)
    table_lin = table_g.reshape(VP, D)
    PERIOD = F * _L // math.gcd(F, _L)
    REP = PERIOD // F
    NPAT = PERIOD // _L
    assert IPW % PERIOD == 0
    off_pat = jnp.tile(offsets.astype(jnp.int32), REP)

    mesh = plsc.VectorSubcoreMesh(core_axis_name="c", subcore_axis_name="s")

    @functools.partial(
        pl.kernel,
        mesh=mesh,
        compiler_params=pltpu.CompilerParams(use_tc_tiling_on_sc=False),
        out_type=jax.ShapeDtypeStruct((B * F, D), jnp.float32),
        scratch_types=[
            pltpu.VMEM((IPW,), jnp.int32),
            pltpu.VMEM((PERIOD,), jnp.int32),
            pltpu.VMEM((GROUP_ROWS, D), jnp.float32),
            pltpu.SemaphoreType.DMA,
        ],
    )
    def _k(ids_hbm, table_hbm, off_hbm, out_hbm, idx_v, off_v, buf_v, sem):
        wid = lax.axis_index("s") * NC + lax.axis_index("c")
        base = wid * IPW

        pltpu.sync_copy(off_hbm, off_v)
        pltpu.sync_copy(ids_hbm.at[pl.ds(base, IPW)], idx_v)

        def add_off(j, _):
            off = off_v[pl.ds(lax.rem(j, NPAT) * _L, _L)]
            sl = pl.ds(j * _L, _L)
            i = idx_v[sl] + off
            c = ((i.astype(jnp.float32) + 0.5)
                 * jnp.float32(1.0 / W)).astype(jnp.int32)
            w = i - c * W
            idx_v[sl] = ((((c >> 2) * W + w) << 2) | (c & 3))
            return _

        lax.fori_loop(0, IPW // _L, add_off, None)

        def group(g, _):
            gbase = g * GROUP_ROWS
            copies = []
            for i in range(SPG):
                cbase = gbase + i * GCH
                copies.append(pltpu.async_copy(
                    table_hbm.at[idx_v.at[pl.ds(cbase, GCH)]],
                    buf_v.at[pl.ds(i * GCH, GCH)],
                    sem,
                ))
            for c in copies:
                c.wait()
            pltpu.sync_copy(buf_v, out_hbm.at[pl.ds(base + gbase, GROUP_ROWS)])
            return _

        lax.fori_loop(0, NGROUP, group, None)

    out = _k(ids_flat, table_lin, off_pat)
    return out.reshape(B, F, D)


# speedup vs baseline: 5.8556x; 1.0606x over previous
import functools
import math

import jax
import jax.numpy as jnp
from jax import lax
from jax.experimental import pallas as pl
from jax.experimental.pallas import tpu as pltpu
from jax.experimental.pallas import tpu_sc as plsc

_L = 16


def _sc_num_workers():
    info = plsc.get_sparse_core_info()
    return info.num_cores, info.num_subcores


def kernel(input_ids, table, offsets):
    B, F = input_ids.shape
    V, D = table.shape
    NC, NS = _sc_num_workers()
    NW = NC * NS
    assert B % NW == 0
    RPW = B // NW
    IPW = RPW * F
    assert IPW % _L == 0

    GCH = 128
    assert IPW % GCH == 0
    NCHUNK = IPW // GCH
    SPG = 13
    assert NCHUNK % SPG == 0
    NGROUP = NCHUNK // SPG
    GROUP_ROWS = SPG * GCH

    ids_flat = input_ids.astype(jnp.int32).reshape(B * F)
    assert V % 4 == 0 and 4 * D == 128 and D == 32
    W = 61 * 128
    lanes_pad = 128 * ((V + 127) // 128)
    n_chunks = (V + W - 1) // W
    assert n_chunks * W == lanes_pad
    n_steps = (n_chunks + 3) // 4
    QP = n_steps * W
    VP = 4 * QP

    def _relayout(i0, i1, i2, i3, out_ref):
        x = jnp.concatenate([i0[...], i1[...], i2[...], i3[...]], axis=0)
        col = lax.broadcasted_iota(jnp.int32, (4 * D, 4 * D), 1)
        row = lax.broadcasted_iota(jnp.int32, (4 * D, 4 * D), 0)
        ident = (col == row).astype(jnp.bfloat16)
        x0 = x.astype(jnp.bfloat16)
        r1 = x - x0.astype(jnp.float32)
        x1 = r1.astype(jnp.bfloat16)
        x2 = (r1 - x1.astype(jnp.float32)).astype(jnp.bfloat16)
        acc = None
        for xk in (x0, x1, x2):
            y = lax.dot_general(xk, ident, (((0,), (0,)), ((), ())),
                                preferred_element_type=jnp.float32)
            acc = y if acc is None else acc + y
        out_ref[...] = acc

    uspec = lambda u: pl.BlockSpec(
        (D, W), lambda j, _u=u: (0, jnp.minimum(4 * j + _u, n_chunks - 1)))
    tableT = table.T
    table_g = pl.pallas_call(
        _relayout,
        grid=(n_steps,),
        in_specs=[uspec(0), uspec(1), uspec(2), uspec(3)],
        out_specs=pl.BlockSpec((W, 4 * D), lambda j: (j, 0)),
        out_shape=jax.ShapeDtypeStruct((QP, 4 * D), jnp.float32),
    )(tableT, tableT, tableT, tableT)
    table_lin = table_g.reshape(VP, D)
    PERIOD = F * _L // math.gcd(F, _L)
    REP = PERIOD // F
    NPAT = PERIOD // _L
    assert IPW % PERIOD == 0
    off_pat = jnp.tile(offsets.astype(jnp.int32), REP)

    mesh = plsc.VectorSubcoreMesh(core_axis_name="c", subcore_axis_name="s")

    @functools.partial(
        pl.kernel,
        mesh=mesh,
        compiler_params=pltpu.CompilerParams(use_tc_tiling_on_sc=False),
        out_type=jax.ShapeDtypeStruct((B * F, D), jnp.float32),
        scratch_types=[
            pltpu.VMEM((IPW,), jnp.int32),
            pltpu.VMEM((PERIOD,), jnp.int32),
            pltpu.VMEM((GROUP_ROWS, D), jnp.float32),
            pltpu.SemaphoreType.DMA,
        ],
    )
    def _k(ids_hbm, table_hbm, off_hbm, out_hbm, idx_v, off_v, buf_v, sem):
        wid = lax.axis_index("s") * NC + lax.axis_index("c")
        base = wid * IPW

        pltpu.sync_copy(off_hbm, off_v)
        pltpu.sync_copy(ids_hbm.at[pl.ds(base, IPW)], idx_v)

        def add_off(j, _):
            off = off_v[pl.ds(lax.rem(j, NPAT) * _L, _L)]
            sl = pl.ds(j * _L, _L)
            i = idx_v[sl] + off
            c = ((i.astype(jnp.float32) + 0.5)
                 * jnp.float32(1.0 / W)).astype(jnp.int32)
            w = i - c * W
            idx_v[sl] = ((((c >> 2) * W + w) << 2) | (c & 3))
            return _

        lax.fori_loop(0, IPW // _L, add_off, None)

        def group(g, _):
            gbase = g * GROUP_ROWS
            copies = []
            for i in range(SPG):
                cbase = gbase + i * GCH
                copies.append(pltpu.async_copy(
                    table_hbm.at[idx_v.at[pl.ds(cbase, GCH)]],
                    buf_v.at[pl.ds(i * GCH, GCH)],
                    sem,
                ))
            for c in copies:
                c.wait()
            pltpu.sync_copy(buf_v, out_hbm.at[pl.ds(base + gbase, GROUP_ROWS)])
            return _

        lax.fori_loop(0, NGROUP, group, None)

    out = _k(ids_flat, table_lin, off_pat)
    return out.reshape(B, F, D)


# speedup vs baseline: 6.1450x; 1.0494x over previous
import functools
import math

import jax
import jax.numpy as jnp
from jax import lax
from jax.experimental import pallas as pl
from jax.experimental.pallas import tpu as pltpu
from jax.experimental.pallas import tpu_sc as plsc

_L = 16


def _sc_num_workers():
    info = plsc.get_sparse_core_info()
    return info.num_cores, info.num_subcores


def kernel(input_ids, table, offsets):
    B, F = input_ids.shape
    V, D = table.shape
    NC, NS = _sc_num_workers()
    NW = NC * NS
    assert B % NW == 0
    RPW = B // NW
    IPW = RPW * F
    assert IPW % _L == 0

    GCH = 128
    assert IPW % GCH == 0
    NCHUNK = IPW // GCH
    SPG = 13
    assert NCHUNK % SPG == 0
    NGROUP = NCHUNK // SPG
    GROUP_ROWS = SPG * GCH

    ids_flat = input_ids.astype(jnp.int32).reshape(B * F)
    assert V % 4 == 0 and 4 * D == 128 and D == 32
    W = 111 * 128
    lanes_pad = 128 * ((V + 127) // 128)
    n_chunks = (V + W - 1) // W
    assert n_chunks * W == lanes_pad
    n_steps = (n_chunks + 3) // 4
    QP = n_steps * W
    VP = 4 * QP

    def _relayout(i0, i1, i2, i3, out_ref):
        x = jnp.concatenate([i0[...], i1[...], i2[...], i3[...]], axis=0)
        col = lax.broadcasted_iota(jnp.int32, (4 * D, 4 * D), 1)
        row = lax.broadcasted_iota(jnp.int32, (4 * D, 4 * D), 0)
        ident = (col == row).astype(jnp.bfloat16)
        x0 = x.astype(jnp.bfloat16)
        r1 = x - x0.astype(jnp.float32)
        x1 = r1.astype(jnp.bfloat16)
        x2 = (r1 - x1.astype(jnp.float32)).astype(jnp.bfloat16)
        acc = None
        for xk in (x0, x1, x2):
            y = lax.dot_general(xk, ident, (((0,), (0,)), ((), ())),
                                preferred_element_type=jnp.float32)
            acc = y if acc is None else acc + y
        out_ref[...] = acc

    uspec = lambda u: pl.BlockSpec(
        (D, W), lambda j, _u=u: (0, jnp.minimum(4 * j + _u, n_chunks - 1)))
    tableT = table.T
    table_g = pl.pallas_call(
        _relayout,
        grid=(n_steps,),
        in_specs=[uspec(0), uspec(1), uspec(2), uspec(3)],
        out_specs=pl.BlockSpec((W, 4 * D), lambda j: (j, 0)),
        out_shape=jax.ShapeDtypeStruct((QP, 4 * D), jnp.float32),
    )(tableT, tableT, tableT, tableT)
    table_lin = table_g.reshape(VP, D)
    PERIOD = F * _L // math.gcd(F, _L)
    REP = PERIOD // F
    NPAT = PERIOD // _L
    assert IPW % PERIOD == 0
    off_pat = jnp.tile(offsets.astype(jnp.int32), REP)

    mesh = plsc.VectorSubcoreMesh(core_axis_name="c", subcore_axis_name="s")

    @functools.partial(
        pl.kernel,
        mesh=mesh,
        compiler_params=pltpu.CompilerParams(use_tc_tiling_on_sc=False),
        out_type=jax.ShapeDtypeStruct((B * F, D), jnp.float32),
        scratch_types=[
            pltpu.VMEM((IPW,), jnp.int32),
            pltpu.VMEM((PERIOD,), jnp.int32),
            pltpu.VMEM((2 * GROUP_ROWS, D), jnp.float32),
            pltpu.SemaphoreType.DMA,
            pltpu.SemaphoreType.DMA,
        ],
    )
    def _k(ids_hbm, table_hbm, off_hbm, out_hbm, idx_v, off_v, buf_v, sem,
           wsem):
        wid = lax.axis_index("s") * NC + lax.axis_index("c")
        base = wid * IPW

        pltpu.sync_copy(off_hbm, off_v)
        pltpu.sync_copy(ids_hbm.at[pl.ds(base, IPW)], idx_v)

        def add_off(j, _):
            off = off_v[pl.ds(lax.rem(j, NPAT) * _L, _L)]
            sl = pl.ds(j * _L, _L)
            i = idx_v[sl] + off
            c = ((i.astype(jnp.float32) + 0.5)
                 * jnp.float32(1.0 / W)).astype(jnp.int32)
            w = i - c * W
            idx_v[sl] = ((((c >> 2) * W + w) << 2) | (c & 3))
            return _

        lax.fori_loop(0, IPW // _L, add_off, None)

        def group(g, _):
            sel = lax.rem(g, 2) * GROUP_ROWS
            gbase = g * GROUP_ROWS

            @pl.when(g >= 2)
            def _drain():
                pltpu.make_async_copy(
                    buf_v.at[pl.ds(sel, GROUP_ROWS)],
                    out_hbm.at[pl.ds(base, GROUP_ROWS)], wsem).wait()

            copies = []
            for i in range(SPG):
                copies.append(pltpu.async_copy(
                    table_hbm.at[idx_v.at[pl.ds(gbase + i * GCH, GCH)]],
                    buf_v.at[pl.ds(sel + i * GCH, GCH)],
                    sem,
                ))
            for c in copies:
                c.wait()
            pltpu.async_copy(buf_v.at[pl.ds(sel, GROUP_ROWS)],
                             out_hbm.at[pl.ds(base + gbase, GROUP_ROWS)], wsem)
            return _

        lax.fori_loop(0, NGROUP, group, None)
        for k in range(min(2, NGROUP)):
            pltpu.make_async_copy(
                buf_v.at[pl.ds(k * GROUP_ROWS, GROUP_ROWS)],
                out_hbm.at[pl.ds(base, GROUP_ROWS)], wsem).wait()

    out = _k(ids_flat, table_lin, off_pat)
    return out.reshape(B, F, D)
